# Initial kernel scaffold; baseline (speedup 1.0000x reference)
import jax
import jax.numpy as jnp
from jax.experimental import pallas as pl


def kernel(x, jx, Wf, bf, Wg, bg, Wh, bh):
    raise NotImplementedError("write your pallas kernel here")


# speedup vs baseline: 3.6267x; 3.6267x over previous
import jax
import jax.numpy as jnp
from jax import lax
from jax.experimental import pallas as pl
from jax.experimental.pallas import tpu as pltpu
from jax.experimental.pallas import tpu_sc as plsc

N = 320000
D = 128
S = 10000

NC = 2
NS = 16


_BN = 2000


def _fg_body(x_ref, wft_ref, bf_ref, wgt_ref, bg_ref, e_ref, p_ref):
    x = x_ref[...]
    f = jnp.dot(x, wft_ref[...], preferred_element_type=jnp.float32) + bf_ref[...]
    g = jnp.dot(x, wgt_ref[...], preferred_element_type=jnp.float32) + bg_ref[...]
    e = jnp.exp(g)
    e_ref[...] = e
    p_ref[...] = f * e


def _fg(x2d, wft, bf2, wgt, bg2):
    grid = (N // _BN,)
    return pl.pallas_call(
        _fg_body,
        grid=grid,
        in_specs=[
            pl.BlockSpec((_BN, D), lambda i: (i, 0)),
            pl.BlockSpec((D, D), lambda i: (0, 0)),
            pl.BlockSpec((1, D), lambda i: (0, 0)),
            pl.BlockSpec((D, D), lambda i: (0, 0)),
            pl.BlockSpec((1, D), lambda i: (0, 0)),
        ],
        out_specs=[
            pl.BlockSpec((_BN, D), lambda i: (i, 0)),
            pl.BlockSpec((_BN, D), lambda i: (i, 0)),
        ],
        out_shape=[
            jax.ShapeDtypeStruct((N, D), jnp.float32),
            jax.ShapeDtypeStruct((N, D), jnp.float32),
        ],
        compiler_params=pltpu.CompilerParams(
            dimension_semantics=("arbitrary",),
        ),
    )(x2d, wft, bf2, wgt, bg2)


_SUB = 40
_CHUNK_SUB = 8
_CHUNK = _SUB * _CHUNK_SUB
_NCHUNK = N // _CHUNK
_NITER = -(-_NCHUNK // NS)
_SSTRIDE = 624
_SROWS = 640


def _segsum_body(e_hbm, p_hbm, jx2_hbm, zeros_hbm, den_hbm, num_hbm,
                 rows_v, idx_v, acc_sh):
    c = lax.axis_index("c")
    s = lax.axis_index("s")

    stripe0 = pl.multiple_of(s * _SSTRIDE, 8)

    pltpu.sync_copy(zeros_hbm, acc_sh.at[pl.ds(stripe0, _SROWS)])
    plsc.subcore_barrier()

    def accumulate(src_hbm):
        def chunk(t, carry):
            cid = s + t * NS

            @pl.when(cid < _NCHUNK)
            def _():
                row0 = pl.multiple_of(cid * _CHUNK, _CHUNK)
                idxrow0 = pl.multiple_of(cid * _CHUNK_SUB, _CHUNK_SUB)
                pltpu.sync_copy(src_hbm.at[pl.ds(row0, _CHUNK)], rows_v)
                pltpu.sync_copy(jx2_hbm.at[pl.ds(idxrow0, _CHUNK_SUB)], idx_v)
                for j in range(_CHUNK_SUB):
                    pltpu.sync_copy(
                        rows_v.at[pl.ds(j * _SUB, _SUB)],
                        acc_sh.at[idx_v.at[j]],
                        add=True,
                    )

            return carry

        lax.fori_loop(0, _NITER, chunk, 0)

    @pl.when(c == 0)
    def _():
        accumulate(e_hbm)

    @pl.when(c == 1)
    def _():
        accumulate(p_hbm)

    plsc.subcore_barrier()

    @pl.when(c == 0)
    def _():
        pltpu.sync_copy(acc_sh.at[pl.ds(stripe0, _SROWS)],
                        den_hbm.at[pl.ds(stripe0, _SROWS)])

    @pl.when(c == 1)
    def _():
        pltpu.sync_copy(acc_sh.at[pl.ds(stripe0, _SROWS)],
                        num_hbm.at[pl.ds(stripe0, _SROWS)])


def _segsum(e, p, jx2, zeros_stripe):
    mesh = plsc.VectorSubcoreMesh(core_axis_name="c", subcore_axis_name="s")
    return pl.kernel(
        _segsum_body,
        out_type=[
            jax.ShapeDtypeStruct((S, D), jnp.float32),
            jax.ShapeDtypeStruct((S, D), jnp.float32),
        ],
        mesh=mesh,
        scratch_types=[
            pltpu.VMEM((_CHUNK, D), jnp.float32),
            pltpu.VMEM((_CHUNK_SUB, _SUB), jnp.int32),
            pltpu.VMEM_SHARED((S, D), jnp.float32),
        ],
    )(e, p, jx2, zeros_stripe)


def _h_body(num_ref, den_ref, wht_ref, bh_ref, h_ref):
    y = num_ref[...] / jnp.maximum(den_ref[...], 1e-30)
    h_ref[...] = (
        jnp.dot(y, wht_ref[...], preferred_element_type=jnp.float32) + bh_ref[...]
    )


def _h(num, den, wht, bh2):
    return pl.pallas_call(
        _h_body,
        out_shape=jax.ShapeDtypeStruct((S, D), jnp.float32),
    )(num, den, wht, bh2)


_GCHUNK = _CHUNK
_GNC = N // _GCHUNK
_GITER = -(-_GNC // (NC * NS))


def _gather_body(h_hbm, jx2_hbm, out_hbm, rows_v, idx_v, sem):
    c = lax.axis_index("c")
    s = lax.axis_index("s")
    w = s * NC + c

    def chunk(t, carry):
        cid = w + t * (NC * NS)

        @pl.when(cid < _GNC)
        def _():
            row0 = pl.multiple_of(cid * _GCHUNK, _GCHUNK)
            idxrow0 = pl.multiple_of(cid * _CHUNK_SUB, _CHUNK_SUB)
            pltpu.sync_copy(jx2_hbm.at[pl.ds(idxrow0, _CHUNK_SUB)],
                            idx_v)
            for j in range(_CHUNK_SUB):
                pltpu.async_copy(
                    h_hbm.at[idx_v.at[j]],
                    rows_v.at[pl.ds(j * _SUB, _SUB)],
                    sem,
                ).wait()
            pltpu.sync_copy(rows_v, out_hbm.at[pl.ds(row0, _GCHUNK)])

        return carry

    lax.fori_loop(0, _GITER, chunk, 0)


def _gather(h, jx2):
    mesh = plsc.VectorSubcoreMesh(core_axis_name="c", subcore_axis_name="s")
    return pl.kernel(
        _gather_body,
        out_type=jax.ShapeDtypeStruct((N, D), jnp.float32),
        mesh=mesh,
        scratch_types=[
            pltpu.VMEM((_GCHUNK, D), jnp.float32),
            pltpu.VMEM((_CHUNK_SUB, _SUB), jnp.int32),
            pltpu.SemaphoreType.DMA,
        ],
    )(h, jx2)


def kernel(x, jx, Wf, bf, Wg, bg, Wh, bh):
    x2d = x.reshape(N, D)
    jx2 = jx.astype(jnp.int32).reshape(N // _SUB, _SUB)
    zeros_stripe = jnp.zeros((_SROWS, D), jnp.float32)

    e, p = _fg(x2d, Wf.T, bf.reshape(1, D), Wg.T, bg.reshape(1, D))
    den, num = _segsum(e, p, jx2, zeros_stripe)
    h = _h(num, den, Wh.T, bh.reshape(1, D))
    out = _gather(h, jx2)
    return out.reshape(1, N, D)


# speedup vs baseline: 5.4021x; 1.4895x over previous
import jax
import jax.numpy as jnp
from jax import lax
from jax.experimental import pallas as pl
from jax.experimental.pallas import tpu as pltpu
from jax.experimental.pallas import tpu_sc as plsc

N = 320000
D = 128
S = 10000

NC = 2
NS = 16


_BN = 2000


def _fg_body(x_ref, wft_ref, bf_ref, wgt_ref, bg_ref, e_ref, p_ref):
    x = x_ref[...]
    f = jnp.dot(x, wft_ref[...], preferred_element_type=jnp.float32) + bf_ref[...]
    g = jnp.dot(x, wgt_ref[...], preferred_element_type=jnp.float32) + bg_ref[...]
    e = jnp.exp(g)
    e_ref[...] = e
    p_ref[...] = f * e


def _fg(x2d, wft, bf2, wgt, bg2):
    grid = (N // _BN,)
    return pl.pallas_call(
        _fg_body,
        grid=grid,
        in_specs=[
            pl.BlockSpec((_BN, D), lambda i: (i, 0)),
            pl.BlockSpec((D, D), lambda i: (0, 0)),
            pl.BlockSpec((1, D), lambda i: (0, 0)),
            pl.BlockSpec((D, D), lambda i: (0, 0)),
            pl.BlockSpec((1, D), lambda i: (0, 0)),
        ],
        out_specs=[
            pl.BlockSpec((_BN, D), lambda i: (i, 0)),
            pl.BlockSpec((_BN, D), lambda i: (i, 0)),
        ],
        out_shape=[
            jax.ShapeDtypeStruct((N, D), jnp.float32),
            jax.ShapeDtypeStruct((N, D), jnp.float32),
        ],
        compiler_params=pltpu.CompilerParams(
            dimension_semantics=("arbitrary",),
        ),
    )(x2d, wft, bf2, wgt, bg2)


_SUB = 40
_CHUNK_SUB = 8
_CHUNK = _SUB * _CHUNK_SUB
_NCHUNK = N // _CHUNK
_NITER = -(-_NCHUNK // NS)
_SSTRIDE = 624
_SROWS = 640


def _segsum_body(e_hbm, p_hbm, jx2_hbm, zeros_hbm, den_hbm, num_hbm,
                 rows_v, idx_v, acc_sh, lsem, ssem):
    c = lax.axis_index("c")
    s = lax.axis_index("s")

    stripe0 = pl.multiple_of(s * _SSTRIDE, 8)

    pltpu.sync_copy(zeros_hbm, acc_sh.at[pl.ds(stripe0, _SROWS)])
    plsc.subcore_barrier()

    def accumulate(src_hbm):
        def chunk(t, carry):
            cid = s + t * NS

            @pl.when(cid < _NCHUNK)
            def _():
                row0 = pl.multiple_of(cid * _CHUNK, _CHUNK)
                idxrow0 = pl.multiple_of(cid * _CHUNK_SUB, _CHUNK_SUB)
                ld_rows = pltpu.async_copy(
                    src_hbm.at[pl.ds(row0, _CHUNK)], rows_v, lsem)
                ld_idx = pltpu.async_copy(
                    jx2_hbm.at[pl.ds(idxrow0, _CHUNK_SUB)], idx_v, lsem)
                ld_rows.wait()
                ld_idx.wait()
                copies = []
                for j in range(_CHUNK_SUB):
                    copies.append(pltpu.async_copy(
                        rows_v.at[pl.ds(j * _SUB, _SUB)],
                        acc_sh.at[idx_v.at[j]],
                        ssem,
                        add=True,
                    ))
                for cp in copies:
                    cp.wait()

            return carry

        lax.fori_loop(0, _NITER, chunk, 0)

    @pl.when(c == 0)
    def _():
        accumulate(e_hbm)

    @pl.when(c == 1)
    def _():
        accumulate(p_hbm)

    plsc.subcore_barrier()

    @pl.when(c == 0)
    def _():
        pltpu.sync_copy(acc_sh.at[pl.ds(stripe0, _SROWS)],
                        den_hbm.at[pl.ds(stripe0, _SROWS)])

    @pl.when(c == 1)
    def _():
        pltpu.sync_copy(acc_sh.at[pl.ds(stripe0, _SROWS)],
                        num_hbm.at[pl.ds(stripe0, _SROWS)])


def _segsum(e, p, jx2, zeros_stripe):
    mesh = plsc.VectorSubcoreMesh(core_axis_name="c", subcore_axis_name="s")
    return pl.kernel(
        _segsum_body,
        out_type=[
            jax.ShapeDtypeStruct((S, D), jnp.float32),
            jax.ShapeDtypeStruct((S, D), jnp.float32),
        ],
        mesh=mesh,
        scratch_types=[
            pltpu.VMEM((_CHUNK, D), jnp.float32),
            pltpu.VMEM((_CHUNK_SUB, _SUB), jnp.int32),
            pltpu.VMEM_SHARED((S, D), jnp.float32),
            pltpu.SemaphoreType.DMA,
            pltpu.SemaphoreType.DMA,
        ],
    )(e, p, jx2, zeros_stripe)


def _h_body(num_ref, den_ref, wht_ref, bh_ref, h_ref):
    y = num_ref[...] / jnp.maximum(den_ref[...], 1e-30)
    h_ref[...] = (
        jnp.dot(y, wht_ref[...], preferred_element_type=jnp.float32) + bh_ref[...]
    )


def _h(num, den, wht, bh2):
    return pl.pallas_call(
        _h_body,
        out_shape=jax.ShapeDtypeStruct((S, D), jnp.float32),
    )(num, den, wht, bh2)


_GCHUNK = _CHUNK
_GNC = N // _GCHUNK
_GITER = -(-_GNC // (NC * NS))


def _gather_body(h_hbm, jx2_hbm, out_hbm, rows_v, idx_v, sem):
    c = lax.axis_index("c")
    s = lax.axis_index("s")
    w = s * NC + c

    def chunk(t, carry):
        cid = w + t * (NC * NS)

        @pl.when(cid < _GNC)
        def _():
            row0 = pl.multiple_of(cid * _GCHUNK, _GCHUNK)
            idxrow0 = pl.multiple_of(cid * _CHUNK_SUB, _CHUNK_SUB)
            pltpu.sync_copy(jx2_hbm.at[pl.ds(idxrow0, _CHUNK_SUB)],
                            idx_v)
            copies = []
            for j in range(_CHUNK_SUB):
                copies.append(pltpu.async_copy(
                    h_hbm.at[idx_v.at[j]],
                    rows_v.at[pl.ds(j * _SUB, _SUB)],
                    sem,
                ))
            for cp in copies:
                cp.wait()
            pltpu.sync_copy(rows_v, out_hbm.at[pl.ds(row0, _GCHUNK)])

        return carry

    lax.fori_loop(0, _GITER, chunk, 0)


def _gather(h, jx2):
    mesh = plsc.VectorSubcoreMesh(core_axis_name="c", subcore_axis_name="s")
    return pl.kernel(
        _gather_body,
        out_type=jax.ShapeDtypeStruct((N, D), jnp.float32),
        mesh=mesh,
        scratch_types=[
            pltpu.VMEM((_GCHUNK, D), jnp.float32),
            pltpu.VMEM((_CHUNK_SUB, _SUB), jnp.int32),
            pltpu.SemaphoreType.DMA,
        ],
    )(h, jx2)


def kernel(x, jx, Wf, bf, Wg, bg, Wh, bh):
    x2d = x.reshape(N, D)
    jx2 = jx.astype(jnp.int32).reshape(N // _SUB, _SUB)
    zeros_stripe = jnp.zeros((_SROWS, D), jnp.float32)

    e, p = _fg(x2d, Wf.T, bf.reshape(1, D), Wg.T, bg.reshape(1, D))
    den, num = _segsum(e, p, jx2, zeros_stripe)
    h = _h(num, den, Wh.T, bh.reshape(1, D))
    out = _gather(h, jx2)
    return out.reshape(1, N, D)


# speedup vs baseline: 5.9191x; 1.0957x over previous
import jax
import jax.numpy as jnp
from jax import lax
from jax.experimental import pallas as pl
from jax.experimental.pallas import tpu as pltpu
from jax.experimental.pallas import tpu_sc as plsc

N = 320000
D = 128
S = 10000

NC = 2
NS = 16
NW = NC * NS


_BN = 2000


def _fg_body(x_ref, wft_ref, bf_ref, wgt_ref, bg_ref, e_ref, p_ref):
    x = x_ref[...]
    f = jnp.dot(x, wft_ref[...], preferred_element_type=jnp.float32) + bf_ref[...]
    g = jnp.dot(x, wgt_ref[...], preferred_element_type=jnp.float32) + bg_ref[...]
    e = jnp.exp(g)
    e_ref[...] = e
    p_ref[...] = f * e


def _fg(x2d, wft, bf2, wgt, bg2):
    grid = (N // _BN,)
    return pl.pallas_call(
        _fg_body,
        grid=grid,
        in_specs=[
            pl.BlockSpec((_BN, D), lambda i: (i, 0)),
            pl.BlockSpec((D, D), lambda i: (0, 0)),
            pl.BlockSpec((1, D), lambda i: (0, 0)),
            pl.BlockSpec((D, D), lambda i: (0, 0)),
            pl.BlockSpec((1, D), lambda i: (0, 0)),
        ],
        out_specs=[
            pl.BlockSpec((_BN, D), lambda i: (i, 0)),
            pl.BlockSpec((_BN, D), lambda i: (i, 0)),
        ],
        out_shape=[
            jax.ShapeDtypeStruct((N, D), jnp.float32),
            jax.ShapeDtypeStruct((N, D), jnp.float32),
        ],
        compiler_params=pltpu.CompilerParams(
            dimension_semantics=("arbitrary",),
        ),
    )(x2d, wft, bf2, wgt, bg2)


_SSUB = 20
_SCS = 8
_SCHUNK = _SSUB * _SCS
_SNCHUNK = N // _SCHUNK
_SPER = _SNCHUNK // NS
_SPAIRS = _SPER // 2
_SSTRIDE = 624
_SROWS = 640


def _segsum_body(e_hbm, p_hbm, jxs_hbm, zeros_hbm, den_hbm, num_hbm,
                 rows0, rows1, idx0, idx1, acc_sh,
                 lsem0, lsem1, ssem0, ssem1):
    c = lax.axis_index("c")
    s = lax.axis_index("s")
    rows = (rows0, rows1)
    idx = (idx0, idx1)
    lsem = (lsem0, lsem1)
    ssem = (ssem0, ssem1)

    stripe0 = pl.multiple_of(s * _SSTRIDE, 8)

    pltpu.sync_copy(zeros_hbm, acc_sh.at[pl.ds(stripe0, _SROWS)])
    plsc.subcore_barrier()

    def drain_scatters(b):
        pltpu.make_async_copy(
            zeros_hbm.at[pl.ds(0, _SCHUNK)], rows[b], ssem[b]).wait()

    def accumulate(src_hbm):
        def load(k, b):
            cid = s + k * NS
            row0 = pl.multiple_of(cid * _SCHUNK, _SCHUNK)
            idxrow0 = pl.multiple_of(cid * _SCS, _SCS)
            lr = pltpu.async_copy(
                src_hbm.at[pl.ds(row0, _SCHUNK)], rows[b], lsem[b])
            li = pltpu.async_copy(
                jxs_hbm.at[pl.ds(idxrow0, _SCS)], idx[b], lsem[b])
            return lr, li

        def scatter(b):
            for j in range(_SCS):
                pltpu.async_copy(
                    rows[b].at[pl.ds(j * _SSUB, _SSUB)],
                    acc_sh.at[idx[b].at[j]],
                    ssem[b],
                    add=True,
                )

        def pair(k2, carry):
            k0 = k2 * 2

            @pl.when(k2 > 0)
            def _():
                drain_scatters(0)

            l0 = load(k0, 0)

            @pl.when(k2 > 0)
            def _():
                drain_scatters(1)

            l1 = load(k0 + 1, 1)
            for cp in l0:
                cp.wait()
            scatter(0)
            for cp in l1:
                cp.wait()
            scatter(1)
            return carry

        lax.fori_loop(0, _SPAIRS, pair, 0)

        drain_scatters(0)
        lf = load(_SPER - 1, 0)
        for cp in lf:
            cp.wait()
        scatter(0)
        drain_scatters(0)
        drain_scatters(1)

    @pl.when(c == 0)
    def _():
        accumulate(e_hbm)

    @pl.when(c == 1)
    def _():
        accumulate(p_hbm)

    plsc.subcore_barrier()

    @pl.when(c == 0)
    def _():
        pltpu.sync_copy(acc_sh.at[pl.ds(stripe0, _SROWS)],
                        den_hbm.at[pl.ds(stripe0, _SROWS)])

    @pl.when(c == 1)
    def _():
        pltpu.sync_copy(acc_sh.at[pl.ds(stripe0, _SROWS)],
                        num_hbm.at[pl.ds(stripe0, _SROWS)])


def _segsum(e, p, jxs, zeros_stripe):
    mesh = plsc.VectorSubcoreMesh(core_axis_name="c", subcore_axis_name="s")
    return pl.kernel(
        _segsum_body,
        out_type=[
            jax.ShapeDtypeStruct((S, D), jnp.float32),
            jax.ShapeDtypeStruct((S, D), jnp.float32),
        ],
        mesh=mesh,
        scratch_types=[
            pltpu.VMEM((_SCHUNK, D), jnp.float32),
            pltpu.VMEM((_SCHUNK, D), jnp.float32),
            pltpu.VMEM((_SCS, _SSUB), jnp.int32),
            pltpu.VMEM((_SCS, _SSUB), jnp.int32),
            pltpu.VMEM_SHARED((S, D), jnp.float32),
            pltpu.SemaphoreType.DMA,
            pltpu.SemaphoreType.DMA,
            pltpu.SemaphoreType.DMA,
            pltpu.SemaphoreType.DMA,
        ],
    )(e, p, jxs, zeros_stripe)


def _h_body(num_ref, den_ref, wht_ref, bh_ref, h_ref):
    y = num_ref[...] / jnp.maximum(den_ref[...], 1e-30)
    h_ref[...] = (
        jnp.dot(y, wht_ref[...], preferred_element_type=jnp.float32) + bh_ref[...]
    )


def _h(num, den, wht, bh2):
    return pl.pallas_call(
        _h_body,
        out_shape=jax.ShapeDtypeStruct((S, D), jnp.float32),
    )(num, den, wht, bh2)


_GSUB = 50
_GCS = 8
_GCHUNK = _GSUB * _GCS
_GNC = N // _GCHUNK
_GPER = _GNC // NW
_GPAIRS = _GPER // 2


def _gather_body(h_hbm, jxg_hbm, out_hbm, rows0, rows1, idx0, idx1,
                 gsem0, gsem1, osem0, osem1):
    c = lax.axis_index("c")
    s = lax.axis_index("s")
    w = s * NC + c
    rows = (rows0, rows1)
    idx = (idx0, idx1)
    gsem = (gsem0, gsem1)
    osem = (osem0, osem1)

    def drain_write(b):
        pltpu.make_async_copy(
            rows[b], out_hbm.at[pl.ds(0, _GCHUNK)], osem[b]).wait()

    def gather_chunk(k, b):
        cid = w + k * NW
        idxrow0 = pl.multiple_of(cid * _GCS, _GCS)
        pltpu.sync_copy(jxg_hbm.at[pl.ds(idxrow0, _GCS)], idx[b])
        copies = []
        for j in range(_GCS):
            copies.append(pltpu.async_copy(
                h_hbm.at[idx[b].at[j]],
                rows[b].at[pl.ds(j * _GSUB, _GSUB)],
                gsem[b],
            ))
        return copies

    def write_back(k, b):
        cid = w + k * NW
        row0 = pl.multiple_of(cid * _GCHUNK, _GCHUNK)
        pltpu.async_copy(rows[b], out_hbm.at[pl.ds(row0, _GCHUNK)], osem[b])

    def pair(k2, carry):
        k0 = k2 * 2

        @pl.when(k2 > 0)
        def _():
            drain_write(0)

        g0 = gather_chunk(k0, 0)

        @pl.when(k2 > 0)
        def _():
            drain_write(1)

        g1 = gather_chunk(k0 + 1, 1)
        for cp in g0:
            cp.wait()
        write_back(k0, 0)
        for cp in g1:
            cp.wait()
        write_back(k0 + 1, 1)
        return carry

    lax.fori_loop(0, _GPAIRS, pair, 0)

    drain_write(0)
    gf = gather_chunk(_GPER - 1, 0)
    for cp in gf:
        cp.wait()
    write_back(_GPER - 1, 0)
    drain_write(0)
    drain_write(1)


def _gather(h, jxg):
    mesh = plsc.VectorSubcoreMesh(core_axis_name="c", subcore_axis_name="s")
    return pl.kernel(
        _gather_body,
        out_type=jax.ShapeDtypeStruct((N, D), jnp.float32),
        mesh=mesh,
        scratch_types=[
            pltpu.VMEM((_GCHUNK, D), jnp.float32),
            pltpu.VMEM((_GCHUNK, D), jnp.float32),
            pltpu.VMEM((_GCS, _GSUB), jnp.int32),
            pltpu.VMEM((_GCS, _GSUB), jnp.int32),
            pltpu.SemaphoreType.DMA,
            pltpu.SemaphoreType.DMA,
            pltpu.SemaphoreType.DMA,
            pltpu.SemaphoreType.DMA,
        ],
    )(h, jxg)


def kernel(x, jx, Wf, bf, Wg, bg, Wh, bh):
    x2d = x.reshape(N, D)
    jx32 = jx.astype(jnp.int32)
    jxs = jx32.reshape(N // _SSUB, _SSUB)
    jxg = jx32.reshape(N // _GSUB, _GSUB)
    zeros_stripe = jnp.zeros((_SROWS, D), jnp.float32)

    e, p = _fg(x2d, Wf.T, bf.reshape(1, D), Wg.T, bg.reshape(1, D))
    den, num = _segsum(e, p, jxs, zeros_stripe)
    h = _h(num, den, Wh.T, bh.reshape(1, D))
    out = _gather(h, jxg)
    return out.reshape(1, N, D)


# speedup vs baseline: 7.7790x; 1.3142x over previous
import jax
import jax.numpy as jnp
from jax import lax
from jax.experimental import pallas as pl
from jax.experimental.pallas import tpu as pltpu
from jax.experimental.pallas import tpu_sc as plsc

N = 320000
D = 128
S = 10000

NC = 2
NS = 16
NW = NC * NS


_BN = 2000


def _fg_body(x_ref, wft_ref, bf_ref, wgt_ref, bg_ref, e_ref, p_ref):
    x = x_ref[...]
    f = jnp.dot(x, wft_ref[...], preferred_element_type=jnp.float32) + bf_ref[...]
    g = jnp.dot(x, wgt_ref[...], preferred_element_type=jnp.float32) + bg_ref[...]
    e = jnp.exp(g)
    e_ref[...] = e
    p_ref[...] = f * e


def _fg(x2d, wft, bf2, wgt, bg2):
    grid = (N // _BN,)
    return pl.pallas_call(
        _fg_body,
        grid=grid,
        in_specs=[
            pl.BlockSpec((_BN, D), lambda i: (i, 0)),
            pl.BlockSpec((D, D), lambda i: (0, 0)),
            pl.BlockSpec((1, D), lambda i: (0, 0)),
            pl.BlockSpec((D, D), lambda i: (0, 0)),
            pl.BlockSpec((1, D), lambda i: (0, 0)),
        ],
        out_specs=[
            pl.BlockSpec((_BN, D), lambda i: (i, 0)),
            pl.BlockSpec((_BN, D), lambda i: (i, 0)),
        ],
        out_shape=[
            jax.ShapeDtypeStruct((N, D), jnp.float32),
            jax.ShapeDtypeStruct((N, D), jnp.float32),
        ],
        compiler_params=pltpu.CompilerParams(
            dimension_semantics=("arbitrary",),
        ),
    )(x2d, wft, bf2, wgt, bg2)


_SSUB = 20
_SCS = 8
_SCHUNK = _SSUB * _SCS
_SNCHUNK = N // _SCHUNK
_SPER = _SNCHUNK // NS
_SPAIRS = _SPER // 2
_SSTRIDE = 624
_SROWS = 640


def _segsum_body(e_hbm, p_hbm, jxs_hbm, zeros_hbm, den_hbm, num_hbm,
                 rows0, rows1, idx0, idx1, acc_sh,
                 lsem0, lsem1, ssem0, ssem1):
    c = lax.axis_index("c")
    s = lax.axis_index("s")
    rows = (rows0, rows1)
    idx = (idx0, idx1)
    lsem = (lsem0, lsem1)
    ssem = (ssem0, ssem1)

    stripe0 = pl.multiple_of(s * _SSTRIDE, 8)

    pltpu.sync_copy(zeros_hbm, acc_sh.at[pl.ds(stripe0, _SROWS)])
    plsc.subcore_barrier()

    def drain_scatters(b):
        pltpu.make_async_copy(
            zeros_hbm.at[pl.ds(0, _SCHUNK)], rows[b], ssem[b]).wait()

    def accumulate(src_hbm):
        def load(k, b):
            cid = s + k * NS
            row0 = pl.multiple_of(cid * _SCHUNK, _SCHUNK)
            idxrow0 = pl.multiple_of(cid * _SCS, _SCS)
            lr = pltpu.async_copy(
                src_hbm.at[pl.ds(row0, _SCHUNK)], rows[b], lsem[b])
            li = pltpu.async_copy(
                jxs_hbm.at[pl.ds(idxrow0, _SCS)], idx[b], lsem[b])
            return lr, li

        def scatter(b):
            for j in range(_SCS):
                pltpu.async_copy(
                    rows[b].at[pl.ds(j * _SSUB, _SSUB)],
                    acc_sh.at[idx[b].at[j]],
                    ssem[b],
                    add=True,
                )

        def pair(k2, carry):
            k0 = k2 * 2

            @pl.when(k2 > 0)
            def _():
                drain_scatters(0)

            l0 = load(k0, 0)

            @pl.when(k2 > 0)
            def _():
                drain_scatters(1)

            l1 = load(k0 + 1, 1)
            for cp in l0:
                cp.wait()
            scatter(0)
            for cp in l1:
                cp.wait()
            scatter(1)
            return carry

        lax.fori_loop(0, _SPAIRS, pair, 0)

        drain_scatters(0)
        lf = load(_SPER - 1, 0)
        for cp in lf:
            cp.wait()
        scatter(0)
        drain_scatters(0)
        drain_scatters(1)

    @pl.when(c == 0)
    def _():
        accumulate(e_hbm)

    @pl.when(c == 1)
    def _():
        accumulate(p_hbm)

    plsc.subcore_barrier()

    @pl.when(c == 0)
    def _():
        pltpu.sync_copy(acc_sh.at[pl.ds(stripe0, _SROWS)],
                        den_hbm.at[pl.ds(stripe0, _SROWS)])

    @pl.when(c == 1)
    def _():
        pltpu.sync_copy(acc_sh.at[pl.ds(stripe0, _SROWS)],
                        num_hbm.at[pl.ds(stripe0, _SROWS)])


def _segsum(e, p, jxs, zeros_stripe):
    mesh = plsc.VectorSubcoreMesh(core_axis_name="c", subcore_axis_name="s")
    return pl.kernel(
        _segsum_body,
        out_type=[
            jax.ShapeDtypeStruct((S, D), jnp.float32),
            jax.ShapeDtypeStruct((S, D), jnp.float32),
        ],
        mesh=mesh,
        scratch_types=[
            pltpu.VMEM((_SCHUNK, D), jnp.float32),
            pltpu.VMEM((_SCHUNK, D), jnp.float32),
            pltpu.VMEM((_SCS, _SSUB), jnp.int32),
            pltpu.VMEM((_SCS, _SSUB), jnp.int32),
            pltpu.VMEM_SHARED((S, D), jnp.float32),
            pltpu.SemaphoreType.DMA,
            pltpu.SemaphoreType.DMA,
            pltpu.SemaphoreType.DMA,
            pltpu.SemaphoreType.DMA,
        ],
    )(e, p, jxs, zeros_stripe)


def _h_body(num_ref, den_ref, wht_ref, bh_ref, h_ref):
    y = num_ref[...] / jnp.maximum(den_ref[...], 1e-30)
    h_ref[...] = (
        jnp.dot(y, wht_ref[...], preferred_element_type=jnp.float32) + bh_ref[...]
    )


def _h(num, den, wht, bh2):
    return pl.pallas_call(
        _h_body,
        out_shape=jax.ShapeDtypeStruct((S, D), jnp.float32),
    )(num, den, wht, bh2)


_GSUB = 20
_GCS = 8
_GCHUNK = _GSUB * _GCS
_GNC = N // _GCHUNK
_GPAIRS = _GNC // NW // 2


def _gather_body(h_hbm, jxg_hbm, out_hbm, rows0, rows1, idx0, idx1, h_sh,
                 gsem0, gsem1, osem0, osem1):
    c = lax.axis_index("c")
    s = lax.axis_index("s")
    w = s * NC + c
    rows = (rows0, rows1)
    idx = (idx0, idx1)
    gsem = (gsem0, gsem1)
    osem = (osem0, osem1)

    stripe0 = pl.multiple_of(s * _SSTRIDE, 8)
    pltpu.sync_copy(h_hbm.at[pl.ds(stripe0, _SROWS)],
                    h_sh.at[pl.ds(stripe0, _SROWS)])
    plsc.subcore_barrier()

    def drain_write(b):
        pltpu.make_async_copy(
            rows[b], out_hbm.at[pl.ds(0, _GCHUNK)], osem[b]).wait()

    def gather_chunk(k, b):
        cid = w + k * NW
        idxrow0 = pl.multiple_of(cid * _GCS, _GCS)
        pltpu.sync_copy(jxg_hbm.at[pl.ds(idxrow0, _GCS)], idx[b])
        copies = []
        for j in range(_GCS):
            copies.append(pltpu.async_copy(
                h_sh.at[idx[b].at[j]],
                rows[b].at[pl.ds(j * _GSUB, _GSUB)],
                gsem[b],
            ))
        return copies

    def write_back(k, b):
        cid = w + k * NW
        row0 = pl.multiple_of(cid * _GCHUNK, _GCHUNK)
        pltpu.async_copy(rows[b], out_hbm.at[pl.ds(row0, _GCHUNK)], osem[b])

    def pair(k2, carry):
        k0 = k2 * 2

        @pl.when(k2 > 0)
        def _():
            drain_write(0)

        g0 = gather_chunk(k0, 0)

        @pl.when(k2 > 0)
        def _():
            drain_write(1)

        g1 = gather_chunk(k0 + 1, 1)
        for cp in g0:
            cp.wait()
        write_back(k0, 0)
        for cp in g1:
            cp.wait()
        write_back(k0 + 1, 1)
        return carry

    lax.fori_loop(0, _GPAIRS, pair, 0)

    @pl.when(w < _GNC - 2 * _GPAIRS * NW)
    def _():
        drain_write(0)
        gf = gather_chunk(2 * _GPAIRS, 0)
        for cp in gf:
            cp.wait()
        write_back(2 * _GPAIRS, 0)

    drain_write(0)
    drain_write(1)


def _gather(h, jxg):
    mesh = plsc.VectorSubcoreMesh(core_axis_name="c", subcore_axis_name="s")
    return pl.kernel(
        _gather_body,
        out_type=jax.ShapeDtypeStruct((N, D), jnp.float32),
        mesh=mesh,
        scratch_types=[
            pltpu.VMEM((_GCHUNK, D), jnp.float32),
            pltpu.VMEM((_GCHUNK, D), jnp.float32),
            pltpu.VMEM((_GCS, _GSUB), jnp.int32),
            pltpu.VMEM((_GCS, _GSUB), jnp.int32),
            pltpu.VMEM_SHARED((S, D), jnp.float32),
            pltpu.SemaphoreType.DMA,
            pltpu.SemaphoreType.DMA,
            pltpu.SemaphoreType.DMA,
            pltpu.SemaphoreType.DMA,
        ],
    )(h, jxg)


def kernel(x, jx, Wf, bf, Wg, bg, Wh, bh):
    x2d = x.reshape(N, D)
    jx32 = jx.astype(jnp.int32)
    jxs = jx32.reshape(N // _SSUB, _SSUB)
    zeros_stripe = jnp.zeros((_SROWS, D), jnp.float32)

    e, p = _fg(x2d, Wf.T, bf.reshape(1, D), Wg.T, bg.reshape(1, D))
    den, num = _segsum(e, p, jxs, zeros_stripe)
    h = _h(num, den, Wh.T, bh.reshape(1, D))
    out = _gather(h, jxs)
    return out.reshape(1, N, D)


# speedup vs baseline: 8.6340x; 1.1099x over previous
import jax
import jax.numpy as jnp
from jax import lax
from jax.experimental import pallas as pl
from jax.experimental.pallas import tpu as pltpu
from jax.experimental.pallas import tpu_sc as plsc

N = 320000
D = 128
S = 10000

NC = 2
NS = 16
NW = NC * NS


_BN = 2000


def _fg_body(x_ref, wft_ref, bf_ref, wgt_ref, bg_ref, e_ref, p_ref):
    x = x_ref[...]
    f = jnp.dot(x, wft_ref[...], preferred_element_type=jnp.float32) + bf_ref[...]
    g = jnp.dot(x, wgt_ref[...], preferred_element_type=jnp.float32) + bg_ref[...]
    e = jnp.exp(g)
    e_ref[...] = e
    p_ref[...] = f * e


_M = N // 2


def _fg(x2d, wft, bf2, wgt, bg2, half):
    grid = (_M // _BN,)
    base = half * (_M // _BN)
    return pl.pallas_call(
        _fg_body,
        grid=grid,
        in_specs=[
            pl.BlockSpec((_BN, D), lambda i: (base + i, 0)),
            pl.BlockSpec((D, D), lambda i: (0, 0)),
            pl.BlockSpec((1, D), lambda i: (0, 0)),
            pl.BlockSpec((D, D), lambda i: (0, 0)),
            pl.BlockSpec((1, D), lambda i: (0, 0)),
        ],
        out_specs=[
            pl.BlockSpec((_BN, D), lambda i: (i, 0)),
            pl.BlockSpec((_BN, D), lambda i: (i, 0)),
        ],
        out_shape=[
            jax.ShapeDtypeStruct((_M, D), jnp.float32),
            jax.ShapeDtypeStruct((_M, D), jnp.float32),
        ],
        compiler_params=pltpu.CompilerParams(
            dimension_semantics=("arbitrary",),
        ),
    )(x2d, wft, bf2, wgt, bg2)


_SSUB = 20
_SCS = 8
_SCHUNK = _SSUB * _SCS
_SNCHUNK = _M // _SCHUNK
_SPAIRS = _SNCHUNK // NS // 2
_SLEFT = _SNCHUNK - 2 * _SPAIRS * NS
_SSTRIDE = 624
_SROWS = 640


def _make_segsum_body(half):
    jxrow_base = half * (_M // _SSUB)

    def body(e_hbm, p_hbm, jxs_hbm, zeros_hbm, den_hbm, num_hbm,
             rows0, rows1, idx0, idx1, acc_sh,
             lsem0, lsem1, ssem0, ssem1):
        _segsum_impl(jxrow_base, e_hbm, p_hbm, jxs_hbm, zeros_hbm, den_hbm,
                     num_hbm, rows0, rows1, idx0, idx1, acc_sh,
                     lsem0, lsem1, ssem0, ssem1)

    return body


def _segsum_impl(jxrow_base, e_hbm, p_hbm, jxs_hbm, zeros_hbm, den_hbm,
                 num_hbm, rows0, rows1, idx0, idx1, acc_sh,
                 lsem0, lsem1, ssem0, ssem1):
    c = lax.axis_index("c")
    s = lax.axis_index("s")
    rows = (rows0, rows1)
    idx = (idx0, idx1)
    lsem = (lsem0, lsem1)
    ssem = (ssem0, ssem1)

    stripe0 = pl.multiple_of(s * _SSTRIDE, 8)

    pltpu.sync_copy(zeros_hbm, acc_sh.at[pl.ds(stripe0, _SROWS)])
    plsc.subcore_barrier()

    def drain_scatters(b):
        pltpu.make_async_copy(
            zeros_hbm.at[pl.ds(0, _SCHUNK)], rows[b], ssem[b]).wait()

    def accumulate(src_hbm):
        def load(k, b):
            cid = s + k * NS
            row0 = pl.multiple_of(cid * _SCHUNK, _SCHUNK)
            idxrow0 = pl.multiple_of(jxrow_base + cid * _SCS, _SCS)
            lr = pltpu.async_copy(
                src_hbm.at[pl.ds(row0, _SCHUNK)], rows[b], lsem[b])
            li = pltpu.async_copy(
                jxs_hbm.at[pl.ds(idxrow0, _SCS)], idx[b], lsem[b])
            return lr, li

        def scatter(b):
            for j in range(_SCS):
                pltpu.async_copy(
                    rows[b].at[pl.ds(j * _SSUB, _SSUB)],
                    acc_sh.at[idx[b].at[j]],
                    ssem[b],
                    add=True,
                )

        def pair(k2, carry):
            k0 = k2 * 2

            @pl.when(k2 > 0)
            def _():
                drain_scatters(0)

            l0 = load(k0, 0)

            @pl.when(k2 > 0)
            def _():
                drain_scatters(1)

            l1 = load(k0 + 1, 1)
            for cp in l0:
                cp.wait()
            scatter(0)
            for cp in l1:
                cp.wait()
            scatter(1)
            return carry

        lax.fori_loop(0, _SPAIRS, pair, 0)

        @pl.when(s < _SLEFT)
        def _():
            drain_scatters(0)
            lf = load(2 * _SPAIRS, 0)
            for cp in lf:
                cp.wait()
            scatter(0)

        drain_scatters(0)
        drain_scatters(1)

    @pl.when(c == 0)
    def _():
        accumulate(e_hbm)

    @pl.when(c == 1)
    def _():
        accumulate(p_hbm)

    plsc.subcore_barrier()

    @pl.when(c == 0)
    def _():
        pltpu.sync_copy(acc_sh.at[pl.ds(stripe0, _SROWS)],
                        den_hbm.at[pl.ds(stripe0, _SROWS)])

    @pl.when(c == 1)
    def _():
        pltpu.sync_copy(acc_sh.at[pl.ds(stripe0, _SROWS)],
                        num_hbm.at[pl.ds(stripe0, _SROWS)])


def _segsum(e, p, jxs, zeros_stripe, half):
    mesh = plsc.VectorSubcoreMesh(core_axis_name="c", subcore_axis_name="s")
    return pl.kernel(
        _make_segsum_body(half),
        out_type=[
            jax.ShapeDtypeStruct((S, D), jnp.float32),
            jax.ShapeDtypeStruct((S, D), jnp.float32),
        ],
        mesh=mesh,
        scratch_types=[
            pltpu.VMEM((_SCHUNK, D), jnp.float32),
            pltpu.VMEM((_SCHUNK, D), jnp.float32),
            pltpu.VMEM((_SCS, _SSUB), jnp.int32),
            pltpu.VMEM((_SCS, _SSUB), jnp.int32),
            pltpu.VMEM_SHARED((S, D), jnp.float32),
            pltpu.SemaphoreType.DMA,
            pltpu.SemaphoreType.DMA,
            pltpu.SemaphoreType.DMA,
            pltpu.SemaphoreType.DMA,
        ],
    )(e, p, jxs, zeros_stripe)


def _h_body(num0_ref, num1_ref, den0_ref, den1_ref, wht_ref, bh_ref, h_ref):
    num = num0_ref[...] + num1_ref[...]
    den = den0_ref[...] + den1_ref[...]
    y = num / jnp.maximum(den, 1e-30)
    h_ref[...] = (
        jnp.dot(y, wht_ref[...], preferred_element_type=jnp.float32) + bh_ref[...]
    )


def _h(num0, num1, den0, den1, wht, bh2):
    return pl.pallas_call(
        _h_body,
        out_shape=jax.ShapeDtypeStruct((S, D), jnp.float32),
    )(num0, num1, den0, den1, wht, bh2)


_GSUB = 20
_GCS = 8
_GCHUNK = _GSUB * _GCS
_GNC = N // _GCHUNK
_GPAIRS = _GNC // NW // 2


def _gather_body(h_hbm, jxg_hbm, out_hbm, rows0, rows1, idx0, idx1, h_sh,
                 gsem0, gsem1, osem0, osem1):
    c = lax.axis_index("c")
    s = lax.axis_index("s")
    w = s * NC + c
    rows = (rows0, rows1)
    idx = (idx0, idx1)
    gsem = (gsem0, gsem1)
    osem = (osem0, osem1)

    stripe0 = pl.multiple_of(s * _SSTRIDE, 8)
    pltpu.sync_copy(h_hbm.at[pl.ds(stripe0, _SROWS)],
                    h_sh.at[pl.ds(stripe0, _SROWS)])
    plsc.subcore_barrier()

    def drain_write(b):
        pltpu.make_async_copy(
            rows[b], out_hbm.at[pl.ds(0, _GCHUNK)], osem[b]).wait()

    def gather_chunk(k, b):
        cid = w + k * NW
        idxrow0 = pl.multiple_of(cid * _GCS, _GCS)
        pltpu.sync_copy(jxg_hbm.at[pl.ds(idxrow0, _GCS)], idx[b])
        copies = []
        for j in range(_GCS):
            copies.append(pltpu.async_copy(
                h_sh.at[idx[b].at[j]],
                rows[b].at[pl.ds(j * _GSUB, _GSUB)],
                gsem[b],
            ))
        return copies

    def write_back(k, b):
        cid = w + k * NW
        row0 = pl.multiple_of(cid * _GCHUNK, _GCHUNK)
        pltpu.async_copy(rows[b], out_hbm.at[pl.ds(row0, _GCHUNK)], osem[b])

    def pair(k2, carry):
        k0 = k2 * 2

        @pl.when(k2 > 0)
        def _():
            drain_write(0)

        g0 = gather_chunk(k0, 0)

        @pl.when(k2 > 0)
        def _():
            drain_write(1)

        g1 = gather_chunk(k0 + 1, 1)
        for cp in g0:
            cp.wait()
        write_back(k0, 0)
        for cp in g1:
            cp.wait()
        write_back(k0 + 1, 1)
        return carry

    lax.fori_loop(0, _GPAIRS, pair, 0)

    @pl.when(w < _GNC - 2 * _GPAIRS * NW)
    def _():
        drain_write(0)
        gf = gather_chunk(2 * _GPAIRS, 0)
        for cp in gf:
            cp.wait()
        write_back(2 * _GPAIRS, 0)

    drain_write(0)
    drain_write(1)


def _gather(h, jxg):
    mesh = plsc.VectorSubcoreMesh(core_axis_name="c", subcore_axis_name="s")
    return pl.kernel(
        _gather_body,
        out_type=jax.ShapeDtypeStruct((N, D), jnp.float32),
        mesh=mesh,
        scratch_types=[
            pltpu.VMEM((_GCHUNK, D), jnp.float32),
            pltpu.VMEM((_GCHUNK, D), jnp.float32),
            pltpu.VMEM((_GCS, _GSUB), jnp.int32),
            pltpu.VMEM((_GCS, _GSUB), jnp.int32),
            pltpu.VMEM_SHARED((S, D), jnp.float32),
            pltpu.SemaphoreType.DMA,
            pltpu.SemaphoreType.DMA,
            pltpu.SemaphoreType.DMA,
            pltpu.SemaphoreType.DMA,
        ],
    )(h, jxg)


def kernel(x, jx, Wf, bf, Wg, bg, Wh, bh):
    x2d = x.reshape(N, D)
    jx32 = jx.astype(jnp.int32)
    jxs = jx32.reshape(N // _SSUB, _SSUB)
    zeros_stripe = jnp.zeros((_SROWS, D), jnp.float32)

    wft, wgt = Wf.T, Wg.T
    bf2, bg2 = bf.reshape(1, D), bg.reshape(1, D)
    e0, p0 = _fg(x2d, wft, bf2, wgt, bg2, 0)
    den0, num0 = _segsum(e0, p0, jxs, zeros_stripe, 0)
    e1, p1 = _fg(x2d, wft, bf2, wgt, bg2, 1)
    den1, num1 = _segsum(e1, p1, jxs, zeros_stripe, 1)
    h = _h(num0, num1, den0, den1, Wh.T, bh.reshape(1, D))
    out = _gather(h, jxs)
    return out.reshape(1, N, D)
